# baseline (speedup 1.0000x reference)
import functools

import jax
import jax.numpy as jnp
from jax import lax
from jax.experimental import pallas as pl
from jax.experimental.pallas import tpu as pltpu
from jax.experimental.pallas import tpu_sc as plsc

N = 4096
D_IN = 1024
D_OUT = 1024
E = 16
TOP_K = 2

TM = 128
PAD = N * TOP_K + E * TM
T = PAD // TM

NC = 2
NS = 16
NW = NC * NS
TPW = N // NW
WSW = 128
CH = 32
NCH = TPW // CH


def _router_plan_body(x_ref, wr_ref, p0_ref, p1_ref, wb0_ref, wb1_ref,
                      te_ref, loss_ref):
    x = x_ref[...]
    logits = jnp.dot(x, wr_ref[...], preferred_element_type=jnp.float32)
    m = jnp.max(logits, axis=1, keepdims=True)
    ex = jnp.exp(logits - m)
    probs = ex / jnp.sum(ex, axis=1, keepdims=True)

    iota = lax.broadcasted_iota(jnp.int32, probs.shape, 1)
    pv1 = jnp.max(probs, axis=1, keepdims=True)
    i1 = jnp.min(jnp.where(probs == pv1, iota, E), axis=1, keepdims=True)
    oh1 = (iota == i1)
    masked = jnp.where(oh1, -jnp.inf, probs)
    pv2 = jnp.max(masked, axis=1, keepdims=True)
    i2 = jnp.min(jnp.where(masked == pv2, iota, E), axis=1, keepdims=True)
    oh2 = (iota == i2)

    wsum = pv1 + pv2
    w1 = pv1 / wsum
    w2 = pv2 / wsum
    wb0_ref[...] = jnp.broadcast_to(w1, (N, WSW))
    wb1_ref[...] = jnp.broadcast_to(w2, (N, WSW))

    ohf1 = oh1.astype(jnp.float32)
    ohf2 = oh2.astype(jnp.float32)
    ohsum = ohf1 + ohf2
    counts = jnp.sum(ohsum, axis=0)
    imp = jnp.sum(probs, axis=0) / N
    loss_ref[...] = (E * jnp.sum(imp * counts) / (N * TOP_K)).reshape(1, 1)

    B = 512
    NB = N // B
    r = lax.broadcasted_iota(jnp.int32, (B, B), 0)
    c = lax.broadcasted_iota(jnp.int32, (B, B), 1)
    tri = (r > c).astype(jnp.float32)
    parts = []
    tots = []
    for bi in range(NB):
        blk = ohsum[bi * B:(bi + 1) * B, :]
        parts.append(jnp.dot(tri, blk, preferred_element_type=jnp.float32))
        tots.append(jnp.sum(blk, axis=0, keepdims=True))
    totm = jnp.concatenate(tots, axis=0)
    rb = lax.broadcasted_iota(jnp.int32, (NB, NB), 0)
    cb = lax.broadcasted_iota(jnp.int32, (NB, NB), 1)
    trib = (rb > cb).astype(jnp.float32)
    offs = jnp.dot(trib, totm, preferred_element_type=jnp.float32)
    S = jnp.concatenate(
        [parts[bi] + offs[bi:bi + 1, :] for bi in range(NB)], axis=0)

    ac = jnp.ceil(counts / TM) * TM
    re16 = lax.broadcasted_iota(jnp.int32, (E, E), 0)
    ce16 = lax.broadcasted_iota(jnp.int32, (E, E), 1)
    sue = (re16 < ce16).astype(jnp.float32)
    start = jnp.dot(ac[None, :], sue,
                    preferred_element_type=jnp.float32)

    base = start + S
    p0_ref[...] = jnp.sum(ohf1 * base, axis=1, keepdims=True).astype(jnp.int32)
    p1_ref[...] = jnp.sum(ohf2 * base, axis=1, keepdims=True).astype(jnp.int32)

    endt = (start + ac[None, :]) / TM
    tcol = lax.broadcasted_iota(jnp.int32, (TM, E), 0).astype(jnp.float32)
    te = jnp.sum((tcol >= endt).astype(jnp.int32), axis=1, keepdims=True)
    te_ref[...] = jnp.minimum(te, E - 1)


def _grouped_matmul_body(te_ref, xs_ref, ws_ref, w_ref, b_ref, ys_ref):
    cw = ws_ref[:, 0:1]
    ys_ref[...] = (jnp.dot(xs_ref[...] * cw, w_ref[0],
                           preferred_element_type=jnp.float32)
                   + cw * b_ref[0])


def _sc_dispatch_body(x_hbm, posd_hbm, wb0_hbm, wb1_hbm, xs_hbm, ws_hbm,
                      idx_v, rows_v, w0_v, w1_v, sem):
    wid = lax.axis_index("s") * NC + lax.axis_index("c")
    pltpu.sync_copy(posd_hbm.at[wid], idx_v)
    for ci in range(NCH):
        base = wid * TPW + ci * CH
        pltpu.sync_copy(x_hbm.at[pl.ds(base, CH)], rows_v)
        pltpu.sync_copy(wb0_hbm.at[pl.ds(base, CH)], w0_v)
        pltpu.sync_copy(wb1_hbm.at[pl.ds(base, CH)], w1_v)
        cps = [
            pltpu.async_copy(rows_v, xs_hbm.at[idx_v.at[2 * ci]], sem),
            pltpu.async_copy(rows_v, xs_hbm.at[idx_v.at[2 * ci + 1]], sem),
            pltpu.async_copy(w0_v, ws_hbm.at[idx_v.at[2 * ci]], sem),
            pltpu.async_copy(w1_v, ws_hbm.at[idx_v.at[2 * ci + 1]], sem),
        ]
        for cp in cps:
            cp.wait()


def _sc_combine_body(ys_hbm, posd_hbm, out_hbm, idx_v, g0_v, g1_v, sem):
    wid = lax.axis_index("s") * NC + lax.axis_index("c")
    pltpu.sync_copy(posd_hbm.at[wid], idx_v)
    for ci in range(NCH):
        cpa = pltpu.async_copy(ys_hbm.at[idx_v.at[2 * ci]], g0_v, sem)
        cpb = pltpu.async_copy(ys_hbm.at[idx_v.at[2 * ci + 1]], g1_v, sem)
        cpa.wait()
        cpb.wait()
        def row_loop(rr, _):
            def add_vec(j, _):
                sl = pl.ds(j * 16, 16)
                g0_v[rr, sl] = g0_v[rr, sl] + g1_v[rr, sl]
                return 0
            lax.fori_loop(0, D_OUT // 16, add_vec, 0, unroll=4)
            return 0
        lax.fori_loop(0, CH, row_loop, 0)
        pltpu.sync_copy(g0_v, out_hbm.at[pl.ds(wid * TPW + ci * CH, CH)])


def kernel(input, Wr, W, b):
    x = input

    p0, p1, wb0, wb1, te2, loss = pl.pallas_call(
        _router_plan_body,
        out_shape=(
            jax.ShapeDtypeStruct((N, 1), jnp.int32),
            jax.ShapeDtypeStruct((N, 1), jnp.int32),
            jax.ShapeDtypeStruct((N, WSW), jnp.float32),
            jax.ShapeDtypeStruct((N, WSW), jnp.float32),
            jax.ShapeDtypeStruct((TM, 1), jnp.int32),
            jax.ShapeDtypeStruct((1, 1), jnp.float32),
        ),
    )(x, Wr)

    te = te2.reshape(-1)[:T]
    posd = jnp.stack(
        [p0.reshape(NW, NCH, CH), p1.reshape(NW, NCH, CH)], axis=2
    ).reshape(NW, 2 * NCH, CH)

    mesh = plsc.VectorSubcoreMesh(
        core_axis_name="c", subcore_axis_name="s",
        num_cores=NC, num_subcores=NS)

    xs, ws = pl.kernel(
        _sc_dispatch_body,
        out_type=(
            jax.ShapeDtypeStruct((PAD, D_IN), jnp.float32),
            jax.ShapeDtypeStruct((PAD, WSW), jnp.float32),
        ),
        mesh=mesh,
        scratch_types=[
            pltpu.VMEM((2 * NCH, CH), jnp.int32),
            pltpu.VMEM((CH, D_IN), jnp.float32),
            pltpu.VMEM((CH, WSW), jnp.float32),
            pltpu.VMEM((CH, WSW), jnp.float32),
            pltpu.SemaphoreType.DMA,
        ],
    )(x, posd, wb0, wb1)

    ys = pl.pallas_call(
        _grouped_matmul_body,
        grid_spec=pltpu.PrefetchScalarGridSpec(
            num_scalar_prefetch=1,
            grid=(T,),
            in_specs=[
                pl.BlockSpec((TM, D_IN), lambda t, te: (t, 0)),
                pl.BlockSpec((TM, WSW), lambda t, te: (t, 0)),
                pl.BlockSpec((1, D_IN, D_OUT), lambda t, te: (te[t], 0, 0)),
                pl.BlockSpec((1, 1, D_OUT), lambda t, te: (te[t], 0, 0)),
            ],
            out_specs=pl.BlockSpec((TM, D_OUT), lambda t, te: (t, 0)),
        ),
        out_shape=jax.ShapeDtypeStruct((PAD, D_OUT), jnp.float32),
    )(te, xs, ws, W, b.reshape(E, 1, D_OUT))

    out = pl.kernel(
        _sc_combine_body,
        out_type=jax.ShapeDtypeStruct((N, D_OUT), jnp.float32),
        mesh=mesh,
        scratch_types=[
            pltpu.VMEM((2 * NCH, CH), jnp.int32),
            pltpu.VMEM((CH, D_OUT), jnp.float32),
            pltpu.VMEM((CH, D_OUT), jnp.float32),
            pltpu.SemaphoreType.DMA,
        ],
    )(ys, posd)

    return (out, loss[0, 0])


# speedup vs baseline: 1.1869x; 1.1869x over previous
import functools

import jax
import jax.numpy as jnp
from jax import lax
from jax.experimental import pallas as pl
from jax.experimental.pallas import tpu as pltpu
from jax.experimental.pallas import tpu_sc as plsc

N = 4096
D_IN = 1024
D_OUT = 1024
E = 16
TOP_K = 2

TM = 256
PAD = N * TOP_K + E * TM
T = PAD // TM

NC = 2
NS = 16
NW = NC * NS
TPW = N // NW
WSW = 128
CH = 32
NCH = TPW // CH
CHC = 16
NCHC = TPW // CHC


def _router_plan_body(x_ref, wr_ref, p0_ref, p1_ref, wb0_ref, wb1_ref,
                      te_ref, loss_ref):
    x = x_ref[...]
    logits = jnp.dot(x, wr_ref[...], preferred_element_type=jnp.float32)
    m = jnp.max(logits, axis=1, keepdims=True)
    ex = jnp.exp(logits - m)
    probs = ex / jnp.sum(ex, axis=1, keepdims=True)

    iota = lax.broadcasted_iota(jnp.int32, probs.shape, 1)
    pv1 = jnp.max(probs, axis=1, keepdims=True)
    i1 = jnp.min(jnp.where(probs == pv1, iota, E), axis=1, keepdims=True)
    oh1 = (iota == i1)
    masked = jnp.where(oh1, -jnp.inf, probs)
    pv2 = jnp.max(masked, axis=1, keepdims=True)
    i2 = jnp.min(jnp.where(masked == pv2, iota, E), axis=1, keepdims=True)
    oh2 = (iota == i2)

    wsum = pv1 + pv2
    w1 = pv1 / wsum
    w2 = pv2 / wsum
    wb0_ref[...] = jnp.broadcast_to(w1, (N, WSW))
    wb1_ref[...] = jnp.broadcast_to(w2, (N, WSW))

    ohf1 = oh1.astype(jnp.float32)
    ohf2 = oh2.astype(jnp.float32)
    ohsum = ohf1 + ohf2
    counts = jnp.sum(ohsum, axis=0)
    imp = jnp.sum(probs, axis=0) / N
    loss_ref[...] = (E * jnp.sum(imp * counts) / (N * TOP_K)).reshape(1, 1)

    B = 512
    NB = N // B
    r = lax.broadcasted_iota(jnp.int32, (B, B), 0)
    c = lax.broadcasted_iota(jnp.int32, (B, B), 1)
    tri = (r > c).astype(jnp.float32)
    parts = []
    tots = []
    for bi in range(NB):
        blk = ohsum[bi * B:(bi + 1) * B, :]
        parts.append(jnp.dot(tri, blk, preferred_element_type=jnp.float32))
        tots.append(jnp.sum(blk, axis=0, keepdims=True))
    totm = jnp.concatenate(tots, axis=0)
    rb = lax.broadcasted_iota(jnp.int32, (NB, NB), 0)
    cb = lax.broadcasted_iota(jnp.int32, (NB, NB), 1)
    trib = (rb > cb).astype(jnp.float32)
    offs = jnp.dot(trib, totm, preferred_element_type=jnp.float32)
    S = jnp.concatenate(
        [parts[bi] + offs[bi:bi + 1, :] for bi in range(NB)], axis=0)

    ac = jnp.ceil(counts / TM) * TM
    re16 = lax.broadcasted_iota(jnp.int32, (E, E), 0)
    ce16 = lax.broadcasted_iota(jnp.int32, (E, E), 1)
    sue = (re16 < ce16).astype(jnp.float32)
    start = jnp.dot(ac[None, :], sue,
                    preferred_element_type=jnp.float32)

    base = start + S
    p0_ref[...] = jnp.sum(ohf1 * base, axis=1, keepdims=True).astype(jnp.int32)
    p1_ref[...] = jnp.sum(ohf2 * base, axis=1, keepdims=True).astype(jnp.int32)

    endt = (start + ac[None, :]) / TM
    tcol = lax.broadcasted_iota(jnp.int32, (TM, E), 0).astype(jnp.float32)
    te = jnp.sum((tcol >= endt).astype(jnp.int32), axis=1, keepdims=True)
    te_ref[...] = jnp.minimum(te, E - 1)


def _grouped_matmul_body(te_ref, xs_ref, ws_ref, w_ref, b_ref, ys_ref):
    cw = ws_ref[:, 0:1]
    ys_ref[...] = (jnp.dot(xs_ref[...] * cw, w_ref[0],
                           preferred_element_type=jnp.float32)
                   + cw * b_ref[0])


def _sc_dispatch_body(x_hbm, posd_hbm, wb0_hbm, wb1_hbm, xs_hbm, ws_hbm,
                      idx_v, rows_v, w0_v, w1_v, sem_ld, sem_sc):
    wid = lax.axis_index("s") * NC + lax.axis_index("c")
    pltpu.sync_copy(posd_hbm.at[wid], idx_v)
    lds = [None, None]
    scs = [[], []]

    def issue_load(ci):
        par = ci % 2
        base = wid * TPW + ci * CH
        lds[par] = [
            pltpu.async_copy(x_hbm.at[pl.ds(base, CH)], rows_v.at[par], sem_ld),
            pltpu.async_copy(wb0_hbm.at[pl.ds(base, CH)], w0_v.at[par], sem_ld),
            pltpu.async_copy(wb1_hbm.at[pl.ds(base, CH)], w1_v.at[par], sem_ld),
        ]

    issue_load(0)
    for ci in range(NCH):
        par = ci % 2
        for h in lds[par]:
            h.wait()
        if ci + 1 < NCH:
            for h in scs[1 - par]:
                h.wait()
            issue_load(ci + 1)
        scs[par] = [
            pltpu.async_copy(rows_v.at[par], xs_hbm.at[idx_v.at[2 * ci]],
                             sem_sc),
            pltpu.async_copy(rows_v.at[par], xs_hbm.at[idx_v.at[2 * ci + 1]],
                             sem_sc),
            pltpu.async_copy(w0_v.at[par], ws_hbm.at[idx_v.at[2 * ci]],
                             sem_sc),
            pltpu.async_copy(w1_v.at[par], ws_hbm.at[idx_v.at[2 * ci + 1]],
                             sem_sc),
        ]
    for par in (0, 1):
        for h in scs[par]:
            h.wait()


def _sc_combine_body(ys_hbm, posc_hbm, out_hbm, idx_v, g0_v, g1_v,
                     sem_g, sem_o):
    wid = lax.axis_index("s") * NC + lax.axis_index("c")
    pltpu.sync_copy(posc_hbm.at[wid], idx_v)
    gas = [[], []]
    ows = [None, None]

    def issue_gather(ci):
        par = ci % 2
        gas[par] = [
            pltpu.async_copy(ys_hbm.at[idx_v.at[2 * ci]], g0_v.at[par],
                             sem_g),
            pltpu.async_copy(ys_hbm.at[idx_v.at[2 * ci + 1]], g1_v.at[par],
                             sem_g),
        ]

    issue_gather(0)
    for ci in range(NCHC):
        par = ci % 2
        for h in gas[par]:
            h.wait()
        if ci + 1 < NCHC:
            if ows[1 - par] is not None:
                ows[1 - par].wait()
            issue_gather(ci + 1)

        def row_loop(rr, _, par=par):
            def add_vec(j, _):
                sl = pl.ds(j * 16, 16)
                g0_v[par, rr, sl] = g0_v[par, rr, sl] + g1_v[par, rr, sl]
                return 0
            lax.fori_loop(0, D_OUT // 16, add_vec, 0, unroll=4)
            return 0
        lax.fori_loop(0, CHC, row_loop, 0)
        ows[par] = pltpu.async_copy(
            g0_v.at[par], out_hbm.at[pl.ds(wid * TPW + ci * CHC, CHC)],
            sem_o)
    for par in (0, 1):
        if ows[par] is not None:
            ows[par].wait()


def kernel(input, Wr, W, b):
    x = input

    p0, p1, wb0, wb1, te2, loss = pl.pallas_call(
        _router_plan_body,
        out_shape=(
            jax.ShapeDtypeStruct((N, 1), jnp.int32),
            jax.ShapeDtypeStruct((N, 1), jnp.int32),
            jax.ShapeDtypeStruct((N, WSW), jnp.float32),
            jax.ShapeDtypeStruct((N, WSW), jnp.float32),
            jax.ShapeDtypeStruct((TM, 1), jnp.int32),
            jax.ShapeDtypeStruct((1, 1), jnp.float32),
        ),
    )(x, Wr)

    te = te2.reshape(-1)[:T]
    posd = jnp.stack(
        [p0.reshape(NW, NCH, CH), p1.reshape(NW, NCH, CH)], axis=2
    ).reshape(NW, 2 * NCH, CH)

    mesh = plsc.VectorSubcoreMesh(
        core_axis_name="c", subcore_axis_name="s",
        num_cores=NC, num_subcores=NS)

    xs, ws = pl.kernel(
        _sc_dispatch_body,
        out_type=(
            jax.ShapeDtypeStruct((PAD, D_IN), jnp.float32),
            jax.ShapeDtypeStruct((PAD, WSW), jnp.float32),
        ),
        mesh=mesh,
        scratch_types=[
            pltpu.VMEM((2 * NCH, CH), jnp.int32),
            pltpu.VMEM((2, CH, D_IN), jnp.float32),
            pltpu.VMEM((2, CH, WSW), jnp.float32),
            pltpu.VMEM((2, CH, WSW), jnp.float32),
            pltpu.SemaphoreType.DMA,
            pltpu.SemaphoreType.DMA,
        ],
    )(x, posd, wb0, wb1)

    ys = pl.pallas_call(
        _grouped_matmul_body,
        grid_spec=pltpu.PrefetchScalarGridSpec(
            num_scalar_prefetch=1,
            grid=(T,),
            in_specs=[
                pl.BlockSpec((TM, D_IN), lambda t, te: (t, 0)),
                pl.BlockSpec((TM, WSW), lambda t, te: (t, 0)),
                pl.BlockSpec((1, D_IN, D_OUT), lambda t, te: (te[t], 0, 0)),
                pl.BlockSpec((1, 1, D_OUT), lambda t, te: (te[t], 0, 0)),
            ],
            out_specs=pl.BlockSpec((TM, D_OUT), lambda t, te: (t, 0)),
        ),
        out_shape=jax.ShapeDtypeStruct((PAD, D_OUT), jnp.float32),
    )(te, xs, ws, W, b.reshape(E, 1, D_OUT))

    posc = jnp.stack(
        [p0.reshape(NW, NCHC, CHC), p1.reshape(NW, NCHC, CHC)], axis=2
    ).reshape(NW, 2 * NCHC, CHC)

    out = pl.kernel(
        _sc_combine_body,
        out_type=jax.ShapeDtypeStruct((N, D_OUT), jnp.float32),
        mesh=mesh,
        scratch_types=[
            pltpu.VMEM((2 * NCHC, CHC), jnp.int32),
            pltpu.VMEM((2, CHC, D_OUT), jnp.float32),
            pltpu.VMEM((2, CHC, D_OUT), jnp.float32),
            pltpu.SemaphoreType.DMA,
            pltpu.SemaphoreType.DMA,
        ],
    )(ys, posc)

    return (out, loss[0, 0])
